# Initial kernel scaffold; baseline (speedup 1.0000x reference)
import jax
import jax.numpy as jnp
from jax.experimental import pallas as pl


def kernel(inputs, embedding_weights):
    raise NotImplementedError("write your pallas kernel here")


# speedup vs baseline: 1.7969x; 1.7969x over previous
import functools

import jax
import jax.numpy as jnp
from jax import lax
from jax.experimental import pallas as pl
from jax.experimental.pallas import tpu as pltpu
from jax.experimental.pallas import tpu_sc as plsc

_INFO = plsc.get_sparse_core_info()
_NC = _INFO.num_cores
_NS = _INFO.num_subcores
_NW = _NC * _NS


@functools.partial(jax.jit, static_argnums=(2, 3))
def _sc_gather(table, idx, chunk, n_chunks):
    B = idx.shape[0]
    D = table.shape[1]
    b_per_w = B // _NW
    mesh = plsc.VectorSubcoreMesh(core_axis_name="c", subcore_axis_name="s")

    @functools.partial(
        pl.kernel,
        mesh=mesh,
        compiler_params=pltpu.CompilerParams(use_tc_tiling_on_sc=False),
        out_type=jax.ShapeDtypeStruct((B, D), jnp.float32),
        scratch_types=[
            pltpu.VMEM((chunk,), jnp.int32),
            pltpu.VMEM((chunk, D), jnp.float32),
            pltpu.SemaphoreType.DMA,
        ],
    )
    def k(table_hbm, idx_hbm, out_hbm, idx_v, rows_v, sem):
        wid = lax.axis_index("s") * _NC + lax.axis_index("c")
        base = wid * b_per_w

        def body(i, carry):
            off = base + i * chunk
            pltpu.sync_copy(idx_hbm.at[pl.ds(off, chunk)], idx_v)
            pltpu.async_copy(table_hbm.at[idx_v], rows_v, sem).wait()
            pltpu.sync_copy(rows_v, out_hbm.at[pl.ds(off, chunk)])
            return carry

        lax.fori_loop(0, n_chunks, body, 0)

    return k(table, idx)


def kernel(inputs, embedding_weights):
    B0, B1 = inputs.shape
    V, D = embedding_weights.shape
    B = B0 * B1
    idx_flat = inputs.reshape(B).astype(jnp.int32)
    chunk = 512
    n_chunks = B // (_NW * chunk)
    out = _sc_gather(embedding_weights, idx_flat, chunk, n_chunks)
    return out.reshape(B0, B1, D)


# speedup vs baseline: 1.8778x; 1.0450x over previous
import functools

import jax
import jax.numpy as jnp
from jax import lax
from jax.experimental import pallas as pl
from jax.experimental.pallas import tpu as pltpu
from jax.experimental.pallas import tpu_sc as plsc

_INFO = plsc.get_sparse_core_info()
_NC = _INFO.num_cores
_NS = _INFO.num_subcores
_NW = _NC * _NS


@functools.partial(jax.jit, static_argnums=(2, 3))
def _sc_gather(table, idx, chunk, n_chunks):
    B = idx.shape[0]
    D = table.shape[1]
    b_per_w = B // _NW
    mesh = plsc.VectorSubcoreMesh(core_axis_name="c", subcore_axis_name="s")

    n_pairs = n_chunks // 2

    @functools.partial(
        pl.kernel,
        mesh=mesh,
        compiler_params=pltpu.CompilerParams(use_tc_tiling_on_sc=False),
        out_type=jax.ShapeDtypeStruct((B, D), jnp.float32),
        scratch_types=[
            pltpu.VMEM((b_per_w,), jnp.int32),
            pltpu.VMEM((chunk, D), jnp.float32),
            pltpu.VMEM((chunk, D), jnp.float32),
            pltpu.SemaphoreType.DMA,
            pltpu.SemaphoreType.DMA,
        ],
    )
    def k(table_hbm, idx_hbm, out_hbm, idx_v, rows0, rows1, gsem0, gsem1):
        wid = lax.axis_index("s") * _NC + lax.axis_index("c")
        base = wid * b_per_w

        pltpu.sync_copy(idx_hbm.at[pl.ds(base, b_per_w)], idx_v)

        def gather(i, rows, sem):
            return pltpu.make_async_copy(
                table_hbm.at[idx_v.at[pl.ds(i * chunk, chunk)]], rows, sem)

        gather(0, rows0, gsem0).start()

        def body(j, carry):
            i = j * 2
            gather(i + 1, rows1, gsem1).start()
            gather(i, rows0, gsem0).wait()
            pltpu.sync_copy(rows0, out_hbm.at[pl.ds(base + i * chunk, chunk)])
            @pl.when(j < n_pairs - 1)
            def _():
                gather(i + 2, rows0, gsem0).start()
            gather(i + 1, rows1, gsem1).wait()
            pltpu.sync_copy(
                rows1, out_hbm.at[pl.ds(base + (i + 1) * chunk, chunk)])
            return carry

        lax.fori_loop(0, n_pairs, body, 0)

    return k(table, idx)


def kernel(inputs, embedding_weights):
    B0, B1 = inputs.shape
    V, D = embedding_weights.shape
    B = B0 * B1
    idx_flat = inputs.reshape(B).astype(jnp.int32)
    chunk = 512
    n_chunks = B // (_NW * chunk)
    out = _sc_gather(embedding_weights, idx_flat, chunk, n_chunks)
    return out.reshape(B0, B1, D)
